# baseline (speedup 1.0000x reference)
import functools

import jax
import jax.numpy as jnp
from jax import lax
from jax.experimental import pallas as pl
from jax.experimental.pallas import tpu as pltpu
from jax.experimental.pallas import tpu_sc as plsc

D = 64
H_DEC = 96
TAU = 0.3

_NTILE = 16
_NCORE = 2
_CHUNK = 2048


def _sim_body(f_ref, zg_ref, dxyz_ref, w1a_ref, w1b_ref, w1c_ref, b1_ref,
              w2_ref, b2_ref, s_ref, bmax_ref):
    f = f_ref[...]
    zg = zg_ref[...]
    dx = dxyz_ref[...]
    h = (jnp.dot(f, w1a_ref[...], preferred_element_type=jnp.float32)
         + jnp.dot(zg, w1b_ref[...], preferred_element_type=jnp.float32)
         + jnp.dot(dx, w1c_ref[...], preferred_element_type=jnp.float32)
         + b1_ref[...])
    h = jnp.maximum(h, 0.0)
    s = jnp.dot(h, w2_ref[...], preferred_element_type=jnp.float32) + b2_ref[...]
    s_ref[...] = s
    i = pl.program_id(0)
    local = jnp.max(s)

    @pl.when(i == 0)
    def _():
        bmax_ref[0, 0] = local

    @pl.when(i > 0)
    def _():
        bmax_ref[0, 0] = jnp.maximum(bmax_ref[0, 0], local)


def _sim_scores(f_pts, z_g, delta_xyz, sim_w1, sim_b1, sim_w2, sim_b2):
    n = f_pts.shape[0]
    bn = 4096
    w1a = sim_w1[:D]
    w1b = sim_w1[D:2 * D]
    w1c = sim_w1[2 * D:]
    grid = (n // bn,)
    return pl.pallas_call(
        _sim_body,
        grid=grid,
        in_specs=[
            pl.BlockSpec((bn, D), lambda i: (i, 0)),
            pl.BlockSpec((bn, D), lambda i: (i, 0)),
            pl.BlockSpec((bn, 3), lambda i: (i, 0)),
            pl.BlockSpec((D, D), lambda i: (0, 0)),
            pl.BlockSpec((D, D), lambda i: (0, 0)),
            pl.BlockSpec((3, D), lambda i: (0, 0)),
            pl.BlockSpec((1, D), lambda i: (0, 0)),
            pl.BlockSpec((D, 1), lambda i: (0, 0)),
            pl.BlockSpec((1, 1), lambda i: (0, 0)),
        ],
        out_specs=[pl.BlockSpec((bn, 1), lambda i: (i, 0)),
                   pl.BlockSpec((1, 1), lambda i: (0, 0),
                                memory_space=pltpu.SMEM)],
        out_shape=[jax.ShapeDtypeStruct((n, 1), jnp.float32),
                   jax.ShapeDtypeStruct((1, 1), jnp.float32)],
    )(f_pts, z_g, delta_xyz, w1a, w1b, w1c, sim_b1.reshape(1, D),
      sim_w2, sim_b2.reshape(1, 1))


def _voxel_body(z_ref, tab_ref,
                gw1a_ref, gw1b_ref, gb1_ref, gw2_ref, gb2_ref,
                wih_ref, whh_ref, bih_ref, bhh_ref,
                lng_ref, lnb_ref, fc1_ref, fb1_ref, fc2_ref, fb2_ref,
                fc3_ref, fb3_ref, out_ref):
    z = z_ref[...]
    tab = tab_ref[...]
    msg = tab[:, :D] / jnp.maximum(tab[:, D:D + 1], 1e-30)
    cnt = tab[:, D + 1:D + 2]

    gh = (jnp.dot(z, gw1a_ref[...], preferred_element_type=jnp.float32)
          + jnp.dot(msg, gw1b_ref[...], preferred_element_type=jnp.float32)
          + gb1_ref[...])
    gh = jnp.maximum(gh, 0.0)
    gate = jax.nn.sigmoid(
        jnp.dot(gh, gw2_ref[...], preferred_element_type=jnp.float32)
        + gb2_ref[...])

    gi = jnp.dot(msg, wih_ref[...], preferred_element_type=jnp.float32) + bih_ref[...]
    gh2 = jnp.dot(z, whh_ref[...], preferred_element_type=jnp.float32) + bhh_ref[...]
    i_r = gi[:, :D]
    i_z = gi[:, D:2 * D]
    i_n = gi[:, 2 * D:]
    h_r = gh2[:, :D]
    h_z = gh2[:, D:2 * D]
    h_n = gh2[:, 2 * D:]
    r = jax.nn.sigmoid(i_r + h_r)
    u = jax.nn.sigmoid(i_z + h_z)
    nn_ = jnp.tanh(i_n + r * h_n)
    h_new = (1.0 - u) * nn_ + u * z
    z_cand = z + gate * (h_new - z)
    touched = cnt > 0.0
    z_out = jnp.where(touched, z_cand, z)

    mu = jnp.mean(z_out, axis=-1, keepdims=True)
    var = jnp.mean((z_out - mu) ** 2, axis=-1, keepdims=True)
    xn = (z_out - mu) * jax.lax.rsqrt(var + 1e-5) * lng_ref[...] + lnb_ref[...]
    hd = jnp.maximum(
        jnp.dot(xn, fc1_ref[...], preferred_element_type=jnp.float32)
        + fb1_ref[...], 0.0)
    hd = hd + jnp.maximum(
        jnp.dot(hd, fc2_ref[...], preferred_element_type=jnp.float32)
        + fb2_ref[...], 0.0)
    logit = (jnp.dot(hd, fc3_ref[...], preferred_element_type=jnp.float32)
             + fb3_ref[...])
    occ = jax.nn.sigmoid(logit)

    out_ref[:, :D] = z_out
    out_ref[:, D:] = occ


def _voxel_update(z_latent, tab,
                  gate_w1, gate_b1, gate_w2, gate_b2,
                  gru_wih, gru_whh, gru_bih, gru_bhh,
                  ln_g, ln_b, fc1_w, fc1_b, fc2_w, fc2_b, fc3_w, fc3_b):
    m = z_latent.shape[0]
    bm = 2048
    grid = (m // bm,)
    full = lambda r, c: pl.BlockSpec((r, c), lambda i: (0, 0))
    return pl.pallas_call(
        _voxel_body,
        grid=grid,
        in_specs=[
            pl.BlockSpec((bm, D), lambda i: (i, 0)),
            pl.BlockSpec((bm, 128), lambda i: (i, 0)),
            full(D, D), full(D, D), full(1, D), full(D, 1), full(1, 1),
            full(D, 3 * D), full(D, 3 * D), full(1, 3 * D), full(1, 3 * D),
            full(1, D), full(1, D),
            full(D, H_DEC), full(1, H_DEC), full(H_DEC, H_DEC), full(1, H_DEC),
            full(H_DEC, 1), full(1, 1),
        ],
        out_specs=pl.BlockSpec((bm, D + 1), lambda i: (i, 0)),
        out_shape=jax.ShapeDtypeStruct((m, D + 1), jnp.float32),
    )(z_latent, tab,
      gate_w1[:D], gate_w1[D:], gate_b1.reshape(1, D), gate_w2,
      gate_b2.reshape(1, 1),
      gru_wih.T, gru_whh.T, gru_bih.reshape(1, 3 * D), gru_bhh.reshape(1, 3 * D),
      ln_g.reshape(1, D), ln_b.reshape(1, D),
      fc1_w, fc1_b.reshape(1, H_DEC), fc2_w, fc2_b.reshape(1, H_DEC),
      fc3_w, fc3_b.reshape(1, 1))


_W = 80
_NBKT = 16
_BINCAP = 9216


def _msg_scatter_body(idx_hbm, ef_hbm, zeros_hbm, out_hbm,
                      idx_v, ids_v, rows_v, rows80_v, lidx2d, pids2d,
                      table, cnts, offs, sem):
    m_chunk = 16384
    c = lax.axis_index("c")
    s = lax.axis_index("s")
    n = idx_hbm.shape[0]
    ppw = n // _NTILE

    pltpu.sync_copy(idx_hbm.at[pl.ds(s * ppw, ppw)], idx_v)

    def zids(i, carry):
        ids_v[pl.ds(i * 16, 16)] = jnp.zeros((16,), jnp.int32)
        return carry
    lax.fori_loop(0, _BINCAP // 16, zids, 0)

    nloc = _NBKT // _NCORE
    lanes = lax.iota(jnp.int32, 16)
    for bl in range(nloc):
        def cnt_body(k, off):
            v = idx_v[pl.ds(k * 16, 16)]
            mask = (v >> 14) == jnp.full((16,), c * nloc + bl, jnp.int32)
            mi = jnp.where(mask, jnp.full((16,), 1, jnp.int32),
                           jnp.full((16,), 0, jnp.int32))
            return off + jnp.sum(mi)
        cnts[bl] = lax.fori_loop(0, ppw // 16, cnt_body, 0)

    acc = 0
    for bl in range(nloc):
        offs[bl] = acc
        acc = ((acc + cnts[bl] + 127) >> 7) << 7

    for bl in range(nloc):
        def bin_body(k, off):
            v = idx_v[pl.ds(k * 16, 16)]
            mask = (v >> 14) == jnp.full((16,), c * nloc + bl, jnp.int32)
            mi = jnp.where(mask, jnp.full((16,), 1, jnp.int32),
                           jnp.full((16,), 0, jnp.int32))
            pid = jnp.full((16,), s * ppw + k * 16, jnp.int32) + lanes
            loc = v & jnp.full((16,), 16383, jnp.int32)
            ent = (loc << 17) | pid
            rank = plsc.cumsum(mi) - mi
            tgt = jnp.full((16,), off, jnp.int32) + rank
            plsc.store_scatter(ids_v, [tgt], ent, mask=mask)
            return off + jnp.sum(mi)
        lax.fori_loop(0, ppw // 16, bin_body, offs[bl])

    stripe = m_chunk // _NTILE
    for b_local in range(_NBKT // _NCORE):
        b = c * (_NBKT // _NCORE) + b_local
        cnt_b = cnts[b_local]
        off_b = offs[b_local]
        base = b * m_chunk
        for zi in range(16):
            pltpu.sync_copy(zeros_hbm,
                            table.at[pl.ds(s * stripe + zi * 64, 64), :])

        @pl.when(s == _NTILE - 1)
        def _():
            pltpu.sync_copy(zeros_hbm.at[pl.ds(0, 16), :],
                            table.at[pl.ds(m_chunk, 16), :])

        plsc.subcore_barrier()

        def batch_body(j, carry):
            for k in range(8):
                ent = ids_v[pl.ds(pl.multiple_of(off_b + j * 128 + k * 16, 16),
                                  16)]
                pos = jnp.full((16,), j * 128 + k * 16, jnp.int32) + lanes
                pad = pos >= jnp.full((16,), cnt_b, jnp.int32)
                pid = ent & jnp.full((16,), (1 << 17) - 1, jnp.int32)
                lidx = ent >> 17
                pids2d[0, pl.ds(k * 16, 16)] = jnp.where(
                    pad, jnp.full((16,), 0, jnp.int32), pid)
                lidx2d[0, pl.ds(k * 16, 16)] = jnp.where(
                    pad, jnp.full((16,), m_chunk, jnp.int32), lidx)
            pltpu.async_copy(ef_hbm.at[pids2d.at[0]], rows_v, sem).wait()

            def cp_row(r, carry2):
                for kk in range(_W // 16):
                    rows80_v[r, pl.ds(kk * 16, 16)] = (
                        rows_v[r, pl.ds(kk * 16, 16)])
                return carry2
            lax.fori_loop(0, 128, cp_row, 0)
            pltpu.sync_copy(rows80_v, table.at[lidx2d.at[0]], add=True)
            return carry

        lax.fori_loop(0, (cnt_b + 127) // 128, batch_body, 0)
        plsc.subcore_barrier()

        pltpu.sync_copy(table.at[pl.ds(s * stripe, stripe), :],
                        out_hbm.at[pl.ds(base + s * stripe, stripe),
                                   pl.ds(0, _W)])
        plsc.subcore_barrier()


def _msg_scatter(vox_idx, ef, m):
    n = vox_idx.shape[0]
    mesh = plsc.VectorSubcoreMesh(core_axis_name="c", subcore_axis_name="s")
    f = pl.kernel(
        _msg_scatter_body,
        mesh=mesh,
        compiler_params=pltpu.CompilerParams(needs_layout_passes=False,
                                             use_tc_tiling_on_sc=False),
        out_type=jax.ShapeDtypeStruct((m, 128), jnp.float32),
        scratch_types=[
            pltpu.VMEM((n // _NTILE,), jnp.int32),
            pltpu.VMEM((_BINCAP,), jnp.int32),
            pltpu.VMEM((128, 128), jnp.float32),
            pltpu.VMEM((128, _W), jnp.float32),
            pltpu.VMEM((1, 128), jnp.int32),
            pltpu.VMEM((1, 128), jnp.int32),
            pltpu.VMEM_SHARED((16384 + 16, _W), jnp.float32),
            pltpu.SMEM((_NBKT // _NCORE,), jnp.int32),
            pltpu.SMEM((_NBKT // _NCORE,), jnp.int32),
            pltpu.SemaphoreType.DMA,
        ],
    )
    return f(vox_idx, ef, jnp.zeros((64, _W), jnp.float32))


def kernel(f_pts, z_latent, delta_xyz, vox_idx, sim_w1, sim_b1, sim_w2, sim_b2,
           gate_w1, gate_b1, gate_w2, gate_b2, gru_wih, gru_whh, gru_bih,
           gru_bhh, ln_g, ln_b, fc1_w, fc1_b, fc2_w, fc2_b, fc3_w, fc3_b):
    m = z_latent.shape[0]

    z_g = jnp.take(z_latent, vox_idx, axis=0)
    s, bmax = _sim_scores(f_pts, z_g, delta_xyz, sim_w1, sim_b1, sim_w2, sim_b2)
    s = s[:, 0]

    gmax = jnp.max(bmax)
    e = jnp.exp(jnp.maximum((s - gmax) / TAU, -80.0))

    n = f_pts.shape[0]
    ef_ext = jnp.concatenate(
        [e[:, None] * f_pts, e[:, None], jnp.ones((n, 1), jnp.float32),
         jnp.zeros((n, 128 - D - 2), jnp.float32)], axis=1)
    tab = _msg_scatter(vox_idx, ef_ext, m)

    return _voxel_update(z_latent, tab,
                         gate_w1, gate_b1, gate_w2, gate_b2,
                         gru_wih, gru_whh, gru_bih, gru_bhh,
                         ln_g, ln_b, fc1_w, fc1_b, fc2_w, fc2_b, fc3_w, fc3_b)


# speedup vs baseline: 1.3763x; 1.3763x over previous
import functools

import jax
import jax.numpy as jnp
from jax import lax
from jax.experimental import pallas as pl
from jax.experimental.pallas import tpu as pltpu
from jax.experimental.pallas import tpu_sc as plsc

D = 64
H_DEC = 96
TAU = 0.3

_NTILE = 16
_NCORE = 2
_CHUNK = 2048


def _sim_body(f_ref, zg_ref, dxyz_ref, w1a_ref, w1b_ref, w1c_ref, b1_ref,
              w2_ref, b2_ref, s_ref, bmax_ref):
    f = f_ref[...]
    zg = zg_ref[...]
    dx = dxyz_ref[...]
    h = (jnp.dot(f, w1a_ref[...], preferred_element_type=jnp.float32)
         + jnp.dot(zg, w1b_ref[...], preferred_element_type=jnp.float32)
         + jnp.dot(dx, w1c_ref[...], preferred_element_type=jnp.float32)
         + b1_ref[...])
    h = jnp.maximum(h, 0.0)
    s = jnp.dot(h, w2_ref[...], preferred_element_type=jnp.float32) + b2_ref[...]
    s_ref[...] = s
    i = pl.program_id(0)
    local = jnp.max(s)

    @pl.when(i == 0)
    def _():
        bmax_ref[0, 0] = local

    @pl.when(i > 0)
    def _():
        bmax_ref[0, 0] = jnp.maximum(bmax_ref[0, 0], local)


def _sim_scores(f_pts, z_g, delta_xyz, sim_w1, sim_b1, sim_w2, sim_b2):
    n = f_pts.shape[0]
    bn = 4096
    w1a = sim_w1[:D]
    w1b = sim_w1[D:2 * D]
    w1c = sim_w1[2 * D:]
    grid = (n // bn,)
    return pl.pallas_call(
        _sim_body,
        grid=grid,
        in_specs=[
            pl.BlockSpec((bn, D), lambda i: (i, 0)),
            pl.BlockSpec((bn, D), lambda i: (i, 0)),
            pl.BlockSpec((bn, 3), lambda i: (i, 0)),
            pl.BlockSpec((D, D), lambda i: (0, 0)),
            pl.BlockSpec((D, D), lambda i: (0, 0)),
            pl.BlockSpec((3, D), lambda i: (0, 0)),
            pl.BlockSpec((1, D), lambda i: (0, 0)),
            pl.BlockSpec((D, 1), lambda i: (0, 0)),
            pl.BlockSpec((1, 1), lambda i: (0, 0)),
        ],
        out_specs=[pl.BlockSpec((bn, 1), lambda i: (i, 0)),
                   pl.BlockSpec((1, 1), lambda i: (0, 0),
                                memory_space=pltpu.SMEM)],
        out_shape=[jax.ShapeDtypeStruct((n, 1), jnp.float32),
                   jax.ShapeDtypeStruct((1, 1), jnp.float32)],
    )(f_pts, z_g, delta_xyz, w1a, w1b, w1c, sim_b1.reshape(1, D),
      sim_w2, sim_b2.reshape(1, 1))


def _voxel_body(z_ref, smsg_ref, den_ref, cnt_ref,
                gw1a_ref, gw1b_ref, gb1_ref, gw2_ref, gb2_ref,
                wih_ref, whh_ref, bih_ref, bhh_ref,
                lng_ref, lnb_ref, fc1_ref, fb1_ref, fc2_ref, fb2_ref,
                fc3_ref, fb3_ref, out_ref):
    z = z_ref[...]
    msg = smsg_ref[...] / jnp.maximum(den_ref[...], 1e-30)
    cnt = cnt_ref[...]

    gh = (jnp.dot(z, gw1a_ref[...], preferred_element_type=jnp.float32)
          + jnp.dot(msg, gw1b_ref[...], preferred_element_type=jnp.float32)
          + gb1_ref[...])
    gh = jnp.maximum(gh, 0.0)
    gate = jax.nn.sigmoid(
        jnp.dot(gh, gw2_ref[...], preferred_element_type=jnp.float32)
        + gb2_ref[...])

    gi = jnp.dot(msg, wih_ref[...], preferred_element_type=jnp.float32) + bih_ref[...]
    gh2 = jnp.dot(z, whh_ref[...], preferred_element_type=jnp.float32) + bhh_ref[...]
    i_r = gi[:, :D]
    i_z = gi[:, D:2 * D]
    i_n = gi[:, 2 * D:]
    h_r = gh2[:, :D]
    h_z = gh2[:, D:2 * D]
    h_n = gh2[:, 2 * D:]
    r = jax.nn.sigmoid(i_r + h_r)
    u = jax.nn.sigmoid(i_z + h_z)
    nn_ = jnp.tanh(i_n + r * h_n)
    h_new = (1.0 - u) * nn_ + u * z
    z_cand = z + gate * (h_new - z)
    touched = cnt > 0.0
    z_out = jnp.where(touched, z_cand, z)

    mu = jnp.mean(z_out, axis=-1, keepdims=True)
    var = jnp.mean((z_out - mu) ** 2, axis=-1, keepdims=True)
    xn = (z_out - mu) * jax.lax.rsqrt(var + 1e-5) * lng_ref[...] + lnb_ref[...]
    hd = jnp.maximum(
        jnp.dot(xn, fc1_ref[...], preferred_element_type=jnp.float32)
        + fb1_ref[...], 0.0)
    hd = hd + jnp.maximum(
        jnp.dot(hd, fc2_ref[...], preferred_element_type=jnp.float32)
        + fb2_ref[...], 0.0)
    logit = (jnp.dot(hd, fc3_ref[...], preferred_element_type=jnp.float32)
             + fb3_ref[...])
    occ = jax.nn.sigmoid(logit)

    out_ref[:, :D] = z_out
    out_ref[:, D:] = occ


def _voxel_update(z_latent, s_msg, denom, count,
                  gate_w1, gate_b1, gate_w2, gate_b2,
                  gru_wih, gru_whh, gru_bih, gru_bhh,
                  ln_g, ln_b, fc1_w, fc1_b, fc2_w, fc2_b, fc3_w, fc3_b):
    m = z_latent.shape[0]
    bm = 2048
    grid = (m // bm,)
    full = lambda r, c: pl.BlockSpec((r, c), lambda i: (0, 0))
    return pl.pallas_call(
        _voxel_body,
        grid=grid,
        in_specs=[
            pl.BlockSpec((bm, D), lambda i: (i, 0)),
            pl.BlockSpec((bm, D), lambda i: (i, 0)),
            pl.BlockSpec((bm, 1), lambda i: (i, 0)),
            pl.BlockSpec((bm, 1), lambda i: (i, 0)),
            full(D, D), full(D, D), full(1, D), full(D, 1), full(1, 1),
            full(D, 3 * D), full(D, 3 * D), full(1, 3 * D), full(1, 3 * D),
            full(1, D), full(1, D),
            full(D, H_DEC), full(1, H_DEC), full(H_DEC, H_DEC), full(1, H_DEC),
            full(H_DEC, 1), full(1, 1),
        ],
        out_specs=pl.BlockSpec((bm, D + 1), lambda i: (i, 0)),
        out_shape=jax.ShapeDtypeStruct((m, D + 1), jnp.float32),
    )(z_latent, s_msg, denom, count,
      gate_w1[:D], gate_w1[D:], gate_b1.reshape(1, D), gate_w2,
      gate_b2.reshape(1, 1),
      gru_wih.T, gru_whh.T, gru_bih.reshape(1, 3 * D), gru_bhh.reshape(1, 3 * D),
      ln_g.reshape(1, D), ln_b.reshape(1, D),
      fc1_w, fc1_b.reshape(1, H_DEC), fc2_w, fc2_b.reshape(1, H_DEC),
      fc3_w, fc3_b.reshape(1, 1))


def _dc_scatter_body(idx_hbm, e_hbm, ones_hbm, zeros_hbm, den_out, cnt_out,
                     idx2d, e2d, ones2d, den_t, cnt_t, sem):
    m = den_t.shape[0]
    c = lax.axis_index("c")
    s = lax.axis_index("s")
    nrow = idx_hbm.shape[0]
    rows_per_w = nrow // (_NCORE * _NTILE)
    wid = s * _NCORE + c
    stripe = m // _NTILE

    pltpu.sync_copy(zeros_hbm.at[pl.ds(s * stripe, stripe)],
                    den_t.at[pl.ds(s * stripe, stripe)])
    pltpu.sync_copy(zeros_hbm.at[pl.ds(s * stripe, stripe)],
                    cnt_t.at[pl.ds(s * stripe, stripe)])
    pltpu.sync_copy(ones_hbm, ones2d)
    plsc.subcore_barrier()

    def chunk_body(ci, carry):
        r0 = wid * rows_per_w + ci * 16
        pltpu.sync_copy(idx_hbm.at[pl.ds(r0, 16), :], idx2d)
        pltpu.sync_copy(e_hbm.at[pl.ds(r0, 16), :], e2d)
        hs = []
        for j in range(16):
            hs.append(pltpu.async_copy(e2d.at[j], den_t.at[idx2d.at[j]],
                                       sem, add=True))
            hs.append(pltpu.async_copy(ones2d.at[j], cnt_t.at[idx2d.at[j]],
                                       sem, add=True))
        for h in hs:
            h.wait()
        return carry

    lax.fori_loop(0, rows_per_w // 16, chunk_body, 0)
    plsc.subcore_barrier()

    pltpu.sync_copy(den_t.at[pl.ds(s * stripe, stripe)],
                    den_out.at[c, pl.ds(s * stripe, stripe)])
    pltpu.sync_copy(cnt_t.at[pl.ds(s * stripe, stripe)],
                    cnt_out.at[c, pl.ds(s * stripe, stripe)])


def _dc_scatter(idx2, e2, m):
    ones = jnp.ones((16, 128), jnp.float32)
    zeros = jnp.zeros((m,), jnp.float32)
    mesh = plsc.VectorSubcoreMesh(core_axis_name="c", subcore_axis_name="s")
    f = pl.kernel(
        _dc_scatter_body,
        mesh=mesh,
        out_type=[jax.ShapeDtypeStruct((_NCORE, m), jnp.float32),
                  jax.ShapeDtypeStruct((_NCORE, m), jnp.float32)],
        scratch_types=[
            pltpu.VMEM((16, 128), jnp.int32),
            pltpu.VMEM((16, 128), jnp.float32),
            pltpu.VMEM((16, 128), jnp.float32),
            pltpu.VMEM_SHARED((m,), jnp.float32),
            pltpu.VMEM_SHARED((m,), jnp.float32),
            pltpu.SemaphoreType.DMA,
        ],
    )
    return f(idx2, e2, ones, zeros)


def kernel(f_pts, z_latent, delta_xyz, vox_idx, sim_w1, sim_b1, sim_w2, sim_b2,
           gate_w1, gate_b1, gate_w2, gate_b2, gru_wih, gru_whh, gru_bih,
           gru_bhh, ln_g, ln_b, fc1_w, fc1_b, fc2_w, fc2_b, fc3_w, fc3_b):
    m = z_latent.shape[0]

    z_g = jnp.take(z_latent, vox_idx, axis=0)
    s, bmax = _sim_scores(f_pts, z_g, delta_xyz, sim_w1, sim_b1, sim_w2, sim_b2)
    s = s[:, 0]

    gmax = jnp.max(bmax)
    e = jnp.exp(jnp.maximum((s - gmax) / TAU, -80.0))

    n = f_pts.shape[0]
    den_p, cnt_p = _dc_scatter(vox_idx.reshape(n // 128, 128),
                               e.reshape(n // 128, 128), m)
    denom = den_p[0] + den_p[1]
    count = cnt_p[0] + cnt_p[1]
    s_msg = jax.ops.segment_sum(e[:, None] * f_pts, vox_idx, num_segments=m)

    return _voxel_update(z_latent, s_msg, denom[:, None], count[:, None],
                         gate_w1, gate_b1, gate_w2, gate_b2,
                         gru_wih, gru_whh, gru_bih, gru_bhh,
                         ln_g, ln_b, fc1_w, fc1_b, fc2_w, fc2_b, fc3_w, fc3_b)
